# Initial kernel scaffold; baseline (speedup 1.0000x reference)
import jax
import jax.numpy as jnp
from jax.experimental import pallas as pl


def kernel(edge_index, edge_weight, embedding, W1, b1, W2, b2):
    raise NotImplementedError("write your pallas kernel here")


# speedup vs baseline: 17.2496x; 17.2496x over previous
import functools

import jax
import jax.numpy as jnp
from jax import lax
from jax.experimental import pallas as pl
from jax.experimental.pallas import tpu as pltpu
from jax.experimental.pallas import tpu_sc as plsc

N = 50000
NP = 50176
D_IN = 32
DH = 16
D_HID = 64
E = 800000
NC, NS = 2, 16
NW = NC * NS
ROWS_W = 200
ER = NW * ROWS_W
EP = ER * 128
NODES_S = NP // NS
EDGES_W = ROWS_W * 128
ROWS_S = ER // NS
MC = 100
STG = NODES_S // 4

_MESH = plsc.VectorSubcoreMesh(
    core_axis_name="c", subcore_axis_name="s", num_cores=NC, num_subcores=NS)

_SC_PARAMS = pltpu.CompilerParams(use_tc_tiling_on_sc=False)


@functools.partial(
    pl.kernel,
    out_type=pltpu.HBM((NC * NP,), jnp.float32),
    mesh=_MESH,
    scratch_types=[
        pltpu.VMEM((ROWS_W, 128), jnp.int32),
        pltpu.VMEM((EDGES_W,), jnp.float32),
        pltpu.VMEM_SHARED((NP,), jnp.float32),
    ],
    compiler_params=_SC_PARAMS,
)
def _sc_deg(dst_hbm, ew_hbm, out_hbm, dst_v, ew_v, acc):
    c = lax.axis_index("c")
    s = lax.axis_index("s")
    wid = s * NC + c

    zeros16 = jnp.zeros((16,), jnp.float32)

    def zloop(i, carry):
        ew_v[pl.ds(i * 16, 16)] = zeros16
        return carry

    lax.fori_loop(0, NODES_S // 16, zloop, 0)
    pltpu.sync_copy(ew_v.at[pl.ds(0, NODES_S)],
                    acc.at[pl.ds(s * NODES_S, NODES_S)])
    plsc.subcore_barrier()

    pltpu.sync_copy(dst_hbm.at[pl.ds(wid * ROWS_W, ROWS_W)], dst_v)
    pltpu.sync_copy(ew_hbm.at[pl.ds(wid * EDGES_W, EDGES_W)], ew_v)

    def body(m, carry):
        pltpu.sync_copy(ew_v.at[pl.ds(m * 128, 128)],
                        acc.at[dst_v.at[m]], add=True)
        return carry

    lax.fori_loop(0, ROWS_W, body, 0)
    plsc.subcore_barrier()
    pltpu.sync_copy(acc.at[pl.ds(s * NODES_S, NODES_S)],
                    ew_v.at[pl.ds(0, NODES_S)])
    pltpu.sync_copy(ew_v.at[pl.ds(0, NODES_S)],
                    out_hbm.at[pl.ds(c * NP + s * NODES_S, NODES_S)])


@functools.partial(
    pl.kernel,
    out_type=pltpu.HBM((NC, NP, DH), jnp.float32),
    mesh=_MESH,
    scratch_types=[
        pltpu.VMEM((MC, 128), jnp.int32),
        pltpu.VMEM((MC, 128), jnp.int32),
        pltpu.VMEM((MC, 128), jnp.float32),
        pltpu.VMEM((128, DH), jnp.float32),
        pltpu.VMEM((STG, DH), jnp.float32),
        pltpu.SemaphoreType.DMA,
        pltpu.VMEM_SHARED((NP, DH), jnp.float32),
    ],
    compiler_params=_SC_PARAMS,
)
def _sc_prop(src_hbm, dst_hbm, ew_hbm, g_hbm, out_hbm,
             src_v, dst_v, ew_v, rows_v, stg_v, sem, acc):
    c = lax.axis_index("c")
    s = lax.axis_index("s")

    zeros16 = jnp.zeros((16,), jnp.float32)

    def zloop(i, carry):
        stg_v[i, pl.ds(0, 16)] = zeros16
        return carry

    lax.fori_loop(0, STG, zloop, 0)
    for t in range(4):
        pltpu.sync_copy(stg_v, acc.at[pl.ds(s * NODES_S + t * STG, STG)])
    plsc.subcore_barrier()

    gsrc = g_hbm.at[c]

    def mcbody(mc, carry):
        base = s * ROWS_S + mc * MC
        pltpu.sync_copy(src_hbm.at[pl.ds(base, MC)], src_v)
        pltpu.sync_copy(dst_hbm.at[pl.ds(base, MC)], dst_v)
        pltpu.sync_copy(ew_hbm.at[pl.ds(base, MC)], ew_v)

        def mbody(m, icarry):
            pltpu.async_copy(gsrc.at[src_v.at[m]], rows_v, sem).wait()

            def gbody(g, jcarry):
                e0 = g * 16
                w16 = ew_v[m, pl.ds(e0, 16)]
                for t in range(16):
                    w = w16[t]
                    rows_v[e0 + t, pl.ds(0, 16)] = (
                        rows_v[e0 + t, pl.ds(0, 16)] * w)
                return jcarry

            lax.fori_loop(0, 8, gbody, 0)
            pltpu.sync_copy(rows_v, acc.at[dst_v.at[m]], add=True)
            return icarry

        lax.fori_loop(0, MC, mbody, 0)
        return carry

    lax.fori_loop(0, ROWS_S // MC, mcbody, 0)
    plsc.subcore_barrier()
    for t in range(4):
        pltpu.sync_copy(acc.at[pl.ds(s * NODES_S + t * STG, STG)], stg_v)
        pltpu.sync_copy(stg_v,
                        out_hbm.at[c, pl.ds(s * NODES_S + t * STG, STG)])


BLK = NP // 8


def _dis(degp_ref):
    return lax.rsqrt(1.0 + degp_ref[0, :] + degp_ref[1, :])


def _cat(ref):
    return jnp.concatenate([ref[0], ref[1]], axis=1)


def _tc1_body(degp_ref, x_ref, g0_ref):
    dis = _dis(degp_ref)
    g0 = x_ref[...] * dis[:, None]
    g0_ref[0] = g0[:, :DH]
    g0_ref[1] = g0[:, DH:]


def _tc2_body(degp_ref, s0_ref, g0_ref, w1t_ref, b1_ref, w2t_ref, g2_ref):
    dis = _dis(degp_ref)
    p0 = (_cat(s0_ref) + _cat(g0_ref)) * dis[:, None]
    x1 = jnp.maximum(
        jnp.dot(p0, w1t_ref[...], preferred_element_type=jnp.float32)
        + b1_ref[...][None, :], 0.0)
    h2 = jnp.dot(x1, w2t_ref[...], preferred_element_type=jnp.float32)
    g2 = h2 * dis[:, None]
    g2_ref[0] = g2[:, :DH]
    g2_ref[1] = g2[:, DH:]


def _tc3_body(degp_ref, s2_ref, g2_ref, b2_ref, out_ref):
    dis = _dis(degp_ref)
    out_ref[...] = jnp.maximum(
        (_cat(s2_ref) + _cat(g2_ref)) * dis[:, None]
        + b2_ref[...][None, :], 0.0)


_SPLIT_SPEC = pl.BlockSpec((NC, BLK, DH), lambda i: (0, i, 0))
_ROW_SPEC = pl.BlockSpec((BLK, D_IN), lambda i: (i, 0))
_DEGP_SPEC = pl.BlockSpec((NC, BLK), lambda i: (0, i))


def _tc1(degp, x_p):
    return pl.pallas_call(
        _tc1_body,
        grid=(8,),
        in_specs=[_DEGP_SPEC, _ROW_SPEC],
        out_specs=_SPLIT_SPEC,
        out_shape=jax.ShapeDtypeStruct((NC, NP, DH), jnp.float32),
    )(degp, x_p)


def _tc2(degp, s0, g0, w1t, b1, w2t):
    return pl.pallas_call(
        _tc2_body,
        grid=(8,),
        in_specs=[
            _DEGP_SPEC, _SPLIT_SPEC, _SPLIT_SPEC,
            pl.BlockSpec((D_IN, D_HID), lambda i: (0, 0)),
            pl.BlockSpec((D_HID,), lambda i: (0,)),
            pl.BlockSpec((D_HID, D_IN), lambda i: (0, 0)),
        ],
        out_specs=_SPLIT_SPEC,
        out_shape=jax.ShapeDtypeStruct((NC, NP, DH), jnp.float32),
    )(degp, s0, g0, w1t, b1, w2t)


def _tc3(degp, s2, g2, b2):
    return pl.pallas_call(
        _tc3_body,
        grid=(8,),
        in_specs=[
            _DEGP_SPEC, _SPLIT_SPEC, _SPLIT_SPEC,
            pl.BlockSpec((D_IN,), lambda i: (0,)),
        ],
        out_specs=_ROW_SPEC,
        out_shape=jax.ShapeDtypeStruct((NP, D_IN), jnp.float32),
    )(degp, s2, g2, b2)


def kernel(edge_index, edge_weight, embedding, W1, b1, W2, b2):
    src = edge_index[0].astype(jnp.int32)
    dst = edge_index[1].astype(jnp.int32)
    ew = edge_weight.astype(jnp.float32)
    pad_e = EP - E
    src_p = jnp.concatenate(
        [src, jnp.zeros((pad_e,), jnp.int32)]).reshape(ER, 128)
    dst_p = jnp.concatenate(
        [dst, jnp.zeros((pad_e,), jnp.int32)]).reshape(ER, 128)
    ew_p = jnp.concatenate(
        [ew, jnp.zeros((pad_e,), jnp.float32)]).reshape(ER, 128)
    x_p = jnp.zeros((NP, D_IN), jnp.float32).at[:N].set(embedding)

    degp = _sc_deg(dst_p, ew_p.reshape(EP)).reshape(NC, NP)
    g0 = _tc1(degp, x_p)
    s0 = _sc_prop(src_p, dst_p, ew_p, g0)
    g2 = _tc2(degp, s0, g0, W1.T, b1, W2.T)
    s2 = _sc_prop(src_p, dst_p, ew_p, g2)
    out = _tc3(degp, s2, g2, b2)
    return out[:N]


# speedup vs baseline: 23.3461x; 1.3534x over previous
import functools

import jax
import jax.numpy as jnp
from jax import lax
from jax.experimental import pallas as pl
from jax.experimental.pallas import tpu as pltpu
from jax.experimental.pallas import tpu_sc as plsc

N = 50000
NP = 50176
D_IN = 32
D_HID = 64
E = 800000
NC, NS = 2, 16
NW = NC * NS
ROWS_W = 200
ER = NW * ROWS_W
EP = ER * 128
NODES_S = NP // NS
EDGES_W = ROWS_W * 128
MC = 8
NB = 4
KPF = 2
STG = NODES_S // 28

_MESH = plsc.VectorSubcoreMesh(
    core_axis_name="c", subcore_axis_name="s", num_cores=NC, num_subcores=NS)

_SC_PARAMS = pltpu.CompilerParams(use_tc_tiling_on_sc=False,
                                  internal_scratch_in_bytes=65536)


@functools.partial(
    pl.kernel,
    out_type=pltpu.HBM((NC * NP,), jnp.float32),
    mesh=_MESH,
    scratch_types=[
        pltpu.VMEM((ROWS_W, 128), jnp.int32),
        pltpu.VMEM((EDGES_W,), jnp.float32),
        pltpu.VMEM_SHARED((NP,), jnp.float32),
    ],
    compiler_params=_SC_PARAMS,
)
def _sc_deg(dst_hbm, ew_hbm, out_hbm, dst_v, ew_v, acc):
    c = lax.axis_index("c")
    s = lax.axis_index("s")
    wid = s * NC + c

    zeros16 = jnp.zeros((16,), jnp.float32)

    def zloop(i, carry):
        ew_v[pl.ds(i * 16, 16)] = zeros16
        return carry

    lax.fori_loop(0, NODES_S // 16, zloop, 0)
    pltpu.sync_copy(ew_v.at[pl.ds(0, NODES_S)],
                    acc.at[pl.ds(s * NODES_S, NODES_S)])
    plsc.subcore_barrier()

    pltpu.sync_copy(dst_hbm.at[pl.ds(wid * ROWS_W, ROWS_W)], dst_v)
    pltpu.sync_copy(ew_hbm.at[pl.ds(wid * EDGES_W, EDGES_W)], ew_v)

    def body(m, carry):
        pltpu.sync_copy(ew_v.at[pl.ds(m * 128, 128)],
                        acc.at[dst_v.at[m]], add=True)
        return carry

    lax.fori_loop(0, ROWS_W, body, 0)
    plsc.subcore_barrier()
    pltpu.sync_copy(acc.at[pl.ds(s * NODES_S, NODES_S)],
                    ew_v.at[pl.ds(0, NODES_S)])
    pltpu.sync_copy(ew_v.at[pl.ds(0, NODES_S)],
                    out_hbm.at[pl.ds(c * NP + s * NODES_S, NODES_S)])


@functools.partial(
    pl.kernel,
    out_type=pltpu.HBM((NC, NP, D_IN), jnp.float32),
    mesh=_MESH,
    scratch_types=[
        [pltpu.SemaphoreType.DMA] * NB,
        [pltpu.SemaphoreType.DMA] * NB,
        pltpu.VMEM_SHARED((NP, D_IN), jnp.float32),
    ],
    compiler_params=_SC_PARAMS,
)
def _sc_prop(src_hbm, dst_hbm, ew_hbm, g_hbm, out_hbm, ssems, gsems, acc):
    pl.run_scoped(
        functools.partial(_sc_prop_inner, src_hbm, dst_hbm, ew_hbm, g_hbm,
                          out_hbm, ssems, gsems, acc),
        pltpu.VMEM((MC, 128), jnp.int32),
        pltpu.VMEM((MC, 128), jnp.int32),
        pltpu.VMEM((MC, 128), jnp.float32),
        pltpu.VMEM((NB, 128, D_IN), jnp.float32),
        pltpu.VMEM((STG, D_IN), jnp.float32),
    )


def _sc_prop_inner(src_hbm, dst_hbm, ew_hbm, g_hbm, out_hbm,
                   ssems, gsems, acc, src_v, dst_v, ew_v, rows_v, stg_v):
    c = lax.axis_index("c")
    s = lax.axis_index("s")
    wid = s * NC + c

    zeros16 = jnp.zeros((16,), jnp.float32)

    def zloop(i, carry):
        stg_v[i // 2, pl.ds((i % 2) * 16, 16)] = zeros16
        return carry

    lax.fori_loop(0, STG * 2, zloop, 0)
    for t in range(NODES_S // STG):
        pltpu.sync_copy(stg_v, acc.at[pl.ds(s * NODES_S + t * STG, STG)])
    plsc.subcore_barrier()

    def _fire_gather(m, b):
        pltpu.async_copy(g_hbm.at[src_v.at[m]], rows_v.at[b], gsems[b])

    def _wait_gather(m, b):
        pltpu.make_async_copy(g_hbm.at[src_v.at[m]], rows_v.at[b],
                              gsems[b]).wait()

    def _fire_scatter(m, b):
        pltpu.async_copy(rows_v.at[b], acc.at[dst_v.at[m]], ssems[b],
                         add=True)

    def _wait_scatter(m, b):
        pltpu.make_async_copy(rows_v.at[b], acc.at[dst_v.at[m]],
                              ssems[b]).wait()

    def _scale(m, b):
        buf = rows_v.at[b]

        def gbody(g, jcarry):
            e0 = g * 16
            w16 = ew_v[m, pl.ds(e0, 16)]
            for t in range(16):
                w = w16[t]
                buf[e0 + t, pl.ds(0, 16)] = buf[e0 + t, pl.ds(0, 16)] * w
                buf[e0 + t, pl.ds(16, 16)] = (
                    buf[e0 + t, pl.ds(16, 16)] * w)
            return jcarry

        lax.fori_loop(0, 8, gbody, 0)

    def mcbody(mc_i, carry):
        base = wid * ROWS_W + mc_i * MC
        pltpu.sync_copy(src_hbm.at[pl.ds(base, MC)], src_v)
        pltpu.sync_copy(dst_hbm.at[pl.ds(base, MC)], dst_v)
        pltpu.sync_copy(ew_hbm.at[pl.ds(base, MC)], ew_v)

        for b in range(KPF):
            _fire_gather(b, b)

        def qbody(q, icarry):
            for b in range(NB):
                m = q * NB + b
                _wait_gather(m, b)
                _scale(m, b)
                _fire_scatter(m, b)
                b2 = (b + KPF) % NB
                mnext = m + KPF

                @pl.when(mnext >= NB)
                def _():
                    _wait_scatter(mnext - NB, b2)

                @pl.when(mnext < MC)
                def _():
                    _fire_gather(mnext, b2)
            return icarry

        lax.fori_loop(0, MC // NB, qbody, 0)
        for t in range(NB - KPF):
            m = MC - (NB - KPF) + t
            _wait_scatter(m, m % NB)
        return carry

    lax.fori_loop(0, ROWS_W // MC, mcbody, 0)
    plsc.subcore_barrier()
    for t in range(NODES_S // STG):
        pltpu.sync_copy(acc.at[pl.ds(s * NODES_S + t * STG, STG)], stg_v)
        pltpu.sync_copy(stg_v,
                        out_hbm.at[c, pl.ds(s * NODES_S + t * STG, STG)])


BLK = NP // 8


def _dis(degp_ref):
    return lax.rsqrt(1.0 + degp_ref[0, :] + degp_ref[1, :])


def _tc1_body(degp_ref, x_ref, g0_ref):
    g0_ref[...] = x_ref[...] * _dis(degp_ref)[:, None]


def _tcmid_body(degp_ref, s_ref, g_ref, w1t_ref, b1_ref, w2t_ref, b2_ref,
                gn_ref, out_ref):
    dis = _dis(degp_ref)
    p = (s_ref[0] + s_ref[1] + g_ref[...]) * dis[:, None]
    x1 = jnp.maximum(
        jnp.dot(p, w1t_ref[...], preferred_element_type=jnp.float32)
        + b1_ref[...][None, :], 0.0)
    h2 = jnp.dot(x1, w2t_ref[...], preferred_element_type=jnp.float32)
    gn_ref[...] = h2 * dis[:, None]
    out_ref[...] = jnp.maximum(p + b2_ref[...][None, :], 0.0)


_PART_SPEC = pl.BlockSpec((NC, BLK, D_IN), lambda i: (0, i, 0))
_ROW_SPEC = pl.BlockSpec((BLK, D_IN), lambda i: (i, 0))
_DEGP_SPEC = pl.BlockSpec((NC, BLK), lambda i: (0, i))


def _tc1(degp, x_p):
    return pl.pallas_call(
        _tc1_body,
        grid=(8,),
        in_specs=[_DEGP_SPEC, _ROW_SPEC],
        out_specs=_ROW_SPEC,
        out_shape=jax.ShapeDtypeStruct((NP, D_IN), jnp.float32),
    )(degp, x_p)


def _tc_mid(degp, s_par, g, w1t, b1, w2t, b2):
    return pl.pallas_call(
        _tcmid_body,
        grid=(8,),
        in_specs=[
            _DEGP_SPEC, _PART_SPEC, _ROW_SPEC,
            pl.BlockSpec((D_IN, D_HID), lambda i: (0, 0)),
            pl.BlockSpec((D_HID,), lambda i: (0,)),
            pl.BlockSpec((D_HID, D_IN), lambda i: (0, 0)),
            pl.BlockSpec((D_IN,), lambda i: (0,)),
        ],
        out_specs=(_ROW_SPEC, _ROW_SPEC),
        out_shape=(jax.ShapeDtypeStruct((NP, D_IN), jnp.float32),
                   jax.ShapeDtypeStruct((NP, D_IN), jnp.float32)),
    )(degp, s_par, g, w1t, b1, w2t, b2)


def kernel(edge_index, edge_weight, embedding, W1, b1, W2, b2):
    src = edge_index[0].astype(jnp.int32)
    dst = edge_index[1].astype(jnp.int32)
    ew = edge_weight.astype(jnp.float32)
    pad_e = EP - E
    src_p = jnp.concatenate(
        [src, jnp.zeros((pad_e,), jnp.int32)]).reshape(ER, 128)
    dst_p = jnp.concatenate(
        [dst, jnp.zeros((pad_e,), jnp.int32)]).reshape(ER, 128)
    ew_p = jnp.concatenate(
        [ew, jnp.zeros((pad_e,), jnp.float32)]).reshape(ER, 128)
    x_p = jnp.zeros((NP, D_IN), jnp.float32).at[:N].set(embedding)

    degp = _sc_deg(dst_p, ew_p.reshape(EP)).reshape(NC, NP)
    g0 = _tc1(degp, x_p)
    w1t, w2t = W1.T, W2.T

    nsteps = 2 + jnp.floor(edge_weight[0] * 0.0).astype(jnp.int32)

    def cond(st):
        return st[0] < nsteps

    def step(st):
        i, g, _prev_out = st
        s_par = _sc_prop(src_p, dst_p, ew_p, g)
        g_next, out = _tc_mid(degp, s_par, g, w1t, b1, w2t, b2)
        return (i + 1, g_next, out)

    _, _, out = lax.while_loop(cond, step, (jnp.int32(0), g0, g0))
    return out[:N]


# speedup vs baseline: 28.4616x; 1.2191x over previous
import functools

import jax
import jax.numpy as jnp
from jax import lax
from jax.experimental import pallas as pl
from jax.experimental.pallas import tpu as pltpu
from jax.experimental.pallas import tpu_sc as plsc

N = 50000
NP = 50176
D_IN = 32
DH = 16
D_HID = 64
E = 800000
NC, NS = 2, 16
NW = NC * NS
ROWS_W = 200
ER = NW * ROWS_W
EP = ER * 128
NODES_S = NP // NS
EDGES_W = ROWS_W * 128
ROWS_S = ER // NS
MC = 80
NB = 8
KPF = 4
STG = NODES_S // 4

_MESH = plsc.VectorSubcoreMesh(
    core_axis_name="c", subcore_axis_name="s", num_cores=NC, num_subcores=NS)

_SC_PARAMS = pltpu.CompilerParams(use_tc_tiling_on_sc=False)


@functools.partial(
    pl.kernel,
    out_type=pltpu.HBM((NC * NP,), jnp.float32),
    mesh=_MESH,
    scratch_types=[
        pltpu.VMEM((ROWS_W, 128), jnp.int32),
        pltpu.VMEM((EDGES_W,), jnp.float32),
        pltpu.VMEM_SHARED((NP,), jnp.float32),
    ],
    compiler_params=_SC_PARAMS,
)
def _sc_deg(dst_hbm, ew_hbm, out_hbm, dst_v, ew_v, acc):
    c = lax.axis_index("c")
    s = lax.axis_index("s")
    wid = s * NC + c

    zeros16 = jnp.zeros((16,), jnp.float32)

    def zloop(i, carry):
        ew_v[pl.ds(i * 16, 16)] = zeros16
        return carry

    lax.fori_loop(0, NODES_S // 16, zloop, 0)
    pltpu.sync_copy(ew_v.at[pl.ds(0, NODES_S)],
                    acc.at[pl.ds(s * NODES_S, NODES_S)])
    plsc.subcore_barrier()

    pltpu.sync_copy(dst_hbm.at[pl.ds(wid * ROWS_W, ROWS_W)], dst_v)
    pltpu.sync_copy(ew_hbm.at[pl.ds(wid * EDGES_W, EDGES_W)], ew_v)

    def body(m, carry):
        pltpu.sync_copy(ew_v.at[pl.ds(m * 128, 128)],
                        acc.at[dst_v.at[m]], add=True)
        return carry

    lax.fori_loop(0, ROWS_W, body, 0)
    plsc.subcore_barrier()
    pltpu.sync_copy(acc.at[pl.ds(s * NODES_S, NODES_S)],
                    ew_v.at[pl.ds(0, NODES_S)])
    pltpu.sync_copy(ew_v.at[pl.ds(0, NODES_S)],
                    out_hbm.at[pl.ds(c * NP + s * NODES_S, NODES_S)])


@functools.partial(
    pl.kernel,
    out_type=pltpu.HBM((NC, NP, DH), jnp.float32),
    mesh=_MESH,
    scratch_types=[
        pltpu.VMEM((MC, 128), jnp.int32),
        pltpu.VMEM((MC, 128), jnp.int32),
        pltpu.VMEM((MC, 128), jnp.float32),
        pltpu.VMEM((NB, 128, DH), jnp.float32),
        pltpu.VMEM((STG, DH), jnp.float32),
        [pltpu.SemaphoreType.DMA] * NB,
        [pltpu.SemaphoreType.DMA] * NB,
        pltpu.VMEM_SHARED((NP, DH), jnp.float32),
    ],
    compiler_params=_SC_PARAMS,
)
def _sc_prop(src_hbm, dst_hbm, ew_hbm, g_hbm, out_hbm,
             src_v, dst_v, ew_v, rows_v, stg_v, gsems, ssems, acc):
    c = lax.axis_index("c")
    s = lax.axis_index("s")

    zeros16 = jnp.zeros((16,), jnp.float32)

    def zloop(i, carry):
        stg_v[i, pl.ds(0, 16)] = zeros16
        return carry

    lax.fori_loop(0, STG, zloop, 0)
    for t in range(4):
        pltpu.sync_copy(stg_v, acc.at[pl.ds(s * NODES_S + t * STG, STG)])
    plsc.subcore_barrier()

    gsrc = g_hbm.at[c]

    def _fire_gather(m, b):
        pltpu.async_copy(gsrc.at[src_v.at[m]], rows_v.at[b], gsems[b])

    def _wait_gather(m, b):
        pltpu.make_async_copy(gsrc.at[src_v.at[m]], rows_v.at[b],
                              gsems[b]).wait()

    def _fire_scatter(m, b):
        pltpu.async_copy(rows_v.at[b], acc.at[dst_v.at[m]], ssems[b],
                         add=True)

    def _wait_scatter(m, b):
        pltpu.make_async_copy(rows_v.at[b], acc.at[dst_v.at[m]],
                              ssems[b]).wait()

    def _scale(m, b):
        buf = rows_v.at[b]

        def gbody(g, jcarry):
            e0 = g * 16
            w16 = ew_v[m, pl.ds(e0, 16)]
            for t in range(16):
                w = w16[t]
                buf[e0 + t, pl.ds(0, 16)] = buf[e0 + t, pl.ds(0, 16)] * w
            return jcarry

        lax.fori_loop(0, 8, gbody, 0)

    def mcbody(mc, carry):
        base = s * ROWS_S + mc * MC
        pltpu.sync_copy(src_hbm.at[pl.ds(base, MC)], src_v)
        pltpu.sync_copy(dst_hbm.at[pl.ds(base, MC)], dst_v)
        pltpu.sync_copy(ew_hbm.at[pl.ds(base, MC)], ew_v)

        for b in range(KPF):
            _fire_gather(b, b)

        def qbody(q, icarry):
            for b in range(NB):
                m = q * NB + b
                _wait_gather(m, b)
                _scale(m, b)
                _fire_scatter(m, b)
                b2 = (b + KPF) % NB
                mnext = m + KPF

                @pl.when(mnext >= NB)
                def _():
                    _wait_scatter(mnext - NB, b2)

                @pl.when(mnext < MC)
                def _():
                    _fire_gather(mnext, b2)
            return icarry

        lax.fori_loop(0, MC // NB, qbody, 0)
        for t in range(NB - KPF):
            m = MC - (NB - KPF) + t
            _wait_scatter(m, m % NB)
        return carry

    lax.fori_loop(0, ROWS_S // MC, mcbody, 0)
    plsc.subcore_barrier()
    for t in range(4):
        pltpu.sync_copy(acc.at[pl.ds(s * NODES_S + t * STG, STG)], stg_v)
        pltpu.sync_copy(stg_v,
                        out_hbm.at[c, pl.ds(s * NODES_S + t * STG, STG)])


BLK = NP // 8


def _dis(degp_ref):
    return lax.rsqrt(1.0 + degp_ref[0, :] + degp_ref[1, :])


def _cat(ref):
    return jnp.concatenate([ref[0], ref[1]], axis=1)


def _tc1_body(degp_ref, x_ref, g0_ref):
    dis = _dis(degp_ref)
    g0 = x_ref[...] * dis[:, None]
    g0_ref[0] = g0[:, :DH]
    g0_ref[1] = g0[:, DH:]


def _tc2_body(degp_ref, s0_ref, g0_ref, w1t_ref, b1_ref, w2t_ref, g2_ref):
    dis = _dis(degp_ref)
    p0 = (_cat(s0_ref) + _cat(g0_ref)) * dis[:, None]
    x1 = jnp.maximum(
        jnp.dot(p0, w1t_ref[...], preferred_element_type=jnp.float32)
        + b1_ref[...][None, :], 0.0)
    h2 = jnp.dot(x1, w2t_ref[...], preferred_element_type=jnp.float32)
    g2 = h2 * dis[:, None]
    g2_ref[0] = g2[:, :DH]
    g2_ref[1] = g2[:, DH:]


def _tc3_body(degp_ref, s2_ref, g2_ref, b2_ref, out_ref):
    dis = _dis(degp_ref)
    out_ref[...] = jnp.maximum(
        (_cat(s2_ref) + _cat(g2_ref)) * dis[:, None]
        + b2_ref[...][None, :], 0.0)


_SPLIT_SPEC = pl.BlockSpec((NC, BLK, DH), lambda i: (0, i, 0))
_ROW_SPEC = pl.BlockSpec((BLK, D_IN), lambda i: (i, 0))
_DEGP_SPEC = pl.BlockSpec((NC, BLK), lambda i: (0, i))


def _tc1(degp, x_p):
    return pl.pallas_call(
        _tc1_body,
        grid=(8,),
        in_specs=[_DEGP_SPEC, _ROW_SPEC],
        out_specs=_SPLIT_SPEC,
        out_shape=jax.ShapeDtypeStruct((NC, NP, DH), jnp.float32),
    )(degp, x_p)


def _tc2(degp, s0, g0, w1t, b1, w2t):
    return pl.pallas_call(
        _tc2_body,
        grid=(8,),
        in_specs=[
            _DEGP_SPEC, _SPLIT_SPEC, _SPLIT_SPEC,
            pl.BlockSpec((D_IN, D_HID), lambda i: (0, 0)),
            pl.BlockSpec((D_HID,), lambda i: (0,)),
            pl.BlockSpec((D_HID, D_IN), lambda i: (0, 0)),
        ],
        out_specs=_SPLIT_SPEC,
        out_shape=jax.ShapeDtypeStruct((NC, NP, DH), jnp.float32),
    )(degp, s0, g0, w1t, b1, w2t)


def _tc3(degp, s2, g2, b2):
    return pl.pallas_call(
        _tc3_body,
        grid=(8,),
        in_specs=[
            _DEGP_SPEC, _SPLIT_SPEC, _SPLIT_SPEC,
            pl.BlockSpec((D_IN,), lambda i: (0,)),
        ],
        out_specs=_ROW_SPEC,
        out_shape=jax.ShapeDtypeStruct((NP, D_IN), jnp.float32),
    )(degp, s2, g2, b2)


def kernel(edge_index, edge_weight, embedding, W1, b1, W2, b2):
    src = edge_index[0].astype(jnp.int32)
    dst = edge_index[1].astype(jnp.int32)
    ew = edge_weight.astype(jnp.float32)
    pad_e = EP - E
    src_p = jnp.concatenate(
        [src, jnp.zeros((pad_e,), jnp.int32)]).reshape(ER, 128)
    dst_p = jnp.concatenate(
        [dst, jnp.zeros((pad_e,), jnp.int32)]).reshape(ER, 128)
    ew_p = jnp.concatenate(
        [ew, jnp.zeros((pad_e,), jnp.float32)]).reshape(ER, 128)
    x_p = jnp.zeros((NP, D_IN), jnp.float32).at[:N].set(embedding)

    degp = _sc_deg(dst_p, ew_p.reshape(EP)).reshape(NC, NP)
    g0 = _tc1(degp, x_p)
    s0 = _sc_prop(src_p, dst_p, ew_p, g0)
    g2 = _tc2(degp, s0, g0, W1.T, b1, W2.T)
    s2 = _sc_prop(src_p, dst_p, ew_p, g2)
    out = _tc3(degp, s2, g2, b2)
    return out[:N]
